# baseline (speedup 1.0000x reference)
import dataclasses
import functools

import jax
import jax.numpy as jnp
from jax import lax
from jax.experimental import pallas as pl
from jax.experimental.pallas import tpu as pltpu
from jax.experimental.pallas import tpu_sc as plsc

BM = 128
NJ = 2
GW = 32
CW = 16


def _routing(topk_ids, N, K, E, NB):
    NK = N * K
    ids = topk_ids.reshape(NK).astype(jnp.int32)
    onehot = (ids[:, None] == jnp.arange(E, dtype=jnp.int32)[None, :]).astype(
        jnp.int32)
    csum = jnp.cumsum(onehot, axis=0)
    counts = csum[-1]
    rank = jnp.take_along_axis(csum, ids[:, None], 1)[:, 0] - 1
    blocks_e = (counts + BM - 1) // BM
    bends = jnp.cumsum(blocks_e)
    bstart = bends - blocks_e
    block_expert = jnp.minimum(
        jnp.searchsorted(bends, jnp.arange(NB, dtype=jnp.int32), side="right"),
        E - 1).astype(jnp.int32)
    pp = (bstart[ids] * BM + rank).astype(jnp.int32)
    return block_expert, pp


def _sc_dispatch(hs_packed, pp, N, K, P, D2):
    info = plsc.get_sparse_core_info()
    NC, NS = info.num_cores, info.num_subcores
    NW = NC * NS
    NK = N * K
    per_w = NK // NW
    tok = (jnp.arange(NK, dtype=jnp.int32) // K).astype(jnp.int32)
    mesh = plsc.VectorSubcoreMesh(core_axis_name="c", subcore_axis_name="s")

    @functools.partial(
        pl.kernel, mesh=mesh,
        out_type=jax.ShapeDtypeStruct((P, D2), jnp.int32),
        scratch_types=[pltpu.VMEM((GW,), jnp.int32),
                       pltpu.VMEM((GW,), jnp.int32),
                       pltpu.VMEM((GW, D2), jnp.int32),
                       pltpu.SemaphoreType.DMA])
    def dispatch_kernel(hs_hbm, tok_hbm, pp_hbm, gx_hbm,
                        tok_v, pp_v, rows_v, sem):
        wid = lax.axis_index("s") * NC + lax.axis_index("c")
        base = wid * per_w

        @pl.loop(0, per_w, step=GW)
        def _(c):
            off = base + c
            pltpu.sync_copy(tok_hbm.at[pl.ds(off, GW)], tok_v)
            pltpu.sync_copy(pp_hbm.at[pl.ds(off, GW)], pp_v)
            pltpu.async_copy(hs_hbm.at[tok_v], rows_v, sem).wait()
            pltpu.sync_copy(rows_v, gx_hbm.at[pp_v])

    return dispatch_kernel(hs_packed, tok, pp)


def _tc_grouped_mlp(block_expert, gx, w_up, w_down, N, D, H, E, NB, P,
                    interpret=False):
    HT = H // NJ

    def body(eids_ref, wg_ref, wl_ref, wd_ref, gx_ref, o_ref,
             wg_s, wl_s, wd_s):
        j = pl.program_id(0)
        b = pl.program_id(1)
        new_tile = jnp.logical_or(
            b == 0, eids_ref[b] != eids_ref[jnp.maximum(b - 1, 0)])

        @pl.when(new_tile)
        def _():
            wg_s[...] = wg_ref[0].astype(jnp.bfloat16)
            wl_s[...] = wl_ref[0].astype(jnp.bfloat16)
            wd_s[...] = wd_ref[0].astype(jnp.bfloat16)

        xb = gx_ref[...]
        hg = lax.dot_general(xb, wg_s[...], (((1,), (1,)), ((), ())),
                             preferred_element_type=jnp.float32)
        hl = lax.dot_general(xb, wl_s[...], (((1,), (1,)), ((), ())),
                             preferred_element_type=jnp.float32)
        g = (hg * jax.nn.sigmoid(hg)) * hl
        part = lax.dot_general(g.astype(jnp.bfloat16), wd_s[...],
                               (((1,), (1,)), ((), ())),
                               preferred_element_type=jnp.float32)
        rows = pl.ds(b * BM, BM)

        @pl.when(j == 0)
        def _():
            o_ref[rows, :] = part

        @pl.when(j > 0)
        def _():
            o_ref[rows, :] = o_ref[rows, :] + part

    grid_spec = pltpu.PrefetchScalarGridSpec(
        num_scalar_prefetch=1,
        grid=(NJ, NB),
        in_specs=[
            pl.BlockSpec((1, HT, D), lambda j, b, eids: (eids[b], j, 0)),
            pl.BlockSpec((1, HT, D), lambda j, b, eids: (eids[b], NJ + j, 0)),
            pl.BlockSpec((1, D, HT), lambda j, b, eids: (eids[b], 0, j)),
            pl.BlockSpec((BM, D), lambda j, b, eids: (b, 0)),
        ],
        out_specs=pl.BlockSpec((P, D), lambda j, b, eids: (0, 0)),
        scratch_shapes=[pltpu.VMEM((HT, D), jnp.bfloat16),
                        pltpu.VMEM((HT, D), jnp.bfloat16),
                        pltpu.VMEM((D, HT), jnp.bfloat16)],
    )
    return pl.pallas_call(
        body,
        grid_spec=grid_spec,
        out_shape=jax.ShapeDtypeStruct((P, D), jnp.float32),
        compiler_params=pltpu.CompilerParams(
            dimension_semantics=("arbitrary", "arbitrary"),
            vmem_limit_bytes=100 * 1024 * 1024,
        ),
        interpret=interpret,
    )(block_expert, w_up, w_up, w_down, gx)


def _sc_combine(y, topk_weights, ppN, N, K, D):
    info = plsc.get_sparse_core_info()
    NC, NS = info.num_cores, info.num_subcores
    NW = NC * NS
    per_w = N // NW
    mesh = plsc.VectorSubcoreMesh(core_axis_name="c", subcore_axis_name="s")
    idx0 = ppN[:, 0]
    idx1 = ppN[:, 1]
    twf = topk_weights.reshape(N * K).astype(jnp.float32)

    cp = pltpu.CompilerParams()
    if "needs_layout_passes" in pltpu.CompilerParams.__dataclass_fields__:
        cp = dataclasses.replace(cp, needs_layout_passes=False)

    @functools.partial(
        pl.kernel, mesh=mesh,
        out_type=jax.ShapeDtypeStruct((N, D), jnp.float32),
        compiler_params=cp,
        scratch_types=[pltpu.VMEM((CW,), jnp.int32),
                       pltpu.VMEM((CW,), jnp.int32),
                       pltpu.VMEM((CW * 2,), jnp.float32),
                       pltpu.VMEM((CW, D), jnp.float32),
                       pltpu.VMEM((CW, D), jnp.float32),
                       pltpu.SemaphoreType.DMA])
    def combine_kernel(y_hbm, tw_hbm, i0_hbm, i1_hbm, out_hbm,
                       i0_v, i1_v, tw_v, acc_v, rows_v, sem):
        wid = lax.axis_index("s") * NC + lax.axis_index("c")
        base = wid * per_w

        @pl.loop(0, per_w, step=CW)
        def _(c):
            off = base + c
            pltpu.sync_copy(i0_hbm.at[pl.ds(off, CW)], i0_v)
            pltpu.sync_copy(i1_hbm.at[pl.ds(off, CW)], i1_v)
            pltpu.sync_copy(tw_hbm.at[pl.ds(2 * off, 2 * CW)], tw_v)
            pltpu.async_copy(y_hbm.at[i0_v], acc_v, sem).wait()
            pltpu.async_copy(y_hbm.at[i1_v], rows_v, sem).wait()

            @pl.loop(0, CW)
            def _(t):
                tw0 = plsc.load_gather(
                    tw_v, [jnp.full((16,), 2 * t, jnp.int32)])
                tw1 = plsc.load_gather(
                    tw_v, [jnp.full((16,), 2 * t + 1, jnp.int32)])

                @pl.loop(0, D, step=16)
                def _(d):
                    sl = pl.ds(d, 16)
                    acc_v[t, sl] = (tw0 * acc_v[t, sl]
                                    + tw1 * rows_v[t, sl])

            pltpu.sync_copy(acc_v, out_hbm.at[pl.ds(off, CW)])

    return combine_kernel(y, twf, idx0, idx1)


def kernel(hidden_states, topk_weights, topk_ids, w_up, w_down):
    N, D = hidden_states.shape
    K = topk_ids.shape[1]
    E = w_up.shape[0]
    H = w_down.shape[2]
    NB = (N * K) // BM + E
    P = NB * BM
    D2 = D // 2

    block_expert, pp = _routing(topk_ids, N, K, E, NB)
    hs_packed = lax.bitcast_convert_type(
        hidden_states.astype(jnp.bfloat16).reshape(N, D2, 2), jnp.int32)
    gx_packed = _sc_dispatch(hs_packed, pp, N, K, P, D2)
    gx = lax.bitcast_convert_type(gx_packed, jnp.bfloat16).reshape(P, D)
    y = _tc_grouped_mlp(block_expert, gx, w_up, w_down, N, D, H, E, NB, P)
    return _sc_combine(y, topk_weights, pp.reshape(N, K), N, K, D)


# speedup vs baseline: 1.8294x; 1.8294x over previous
import dataclasses
import functools

import jax
import jax.numpy as jnp
from jax import lax
from jax.experimental import pallas as pl
from jax.experimental.pallas import tpu as pltpu
from jax.experimental.pallas import tpu_sc as plsc

BM = 256
NJ = 4
GW = 32
CW = 32


def _routing(topk_ids, N, K, E, NB):
    NK = N * K
    ids = topk_ids.reshape(NK).astype(jnp.int32)
    onehot = (ids[:, None] == jnp.arange(E, dtype=jnp.int32)[None, :]).astype(
        jnp.int32)
    csum = jnp.cumsum(onehot, axis=0)
    counts = csum[-1]
    rank = jnp.take_along_axis(csum, ids[:, None], 1)[:, 0] - 1
    blocks_e = (counts + BM - 1) // BM
    bends = jnp.cumsum(blocks_e)
    bstart = bends - blocks_e
    block_expert = jnp.minimum(
        jnp.searchsorted(bends, jnp.arange(NB, dtype=jnp.int32), side="right"),
        E - 1).astype(jnp.int32)
    pp = (bstart[ids] * BM + rank).astype(jnp.int32)
    return block_expert, pp


def _sc_dispatch(hidden_states, pp, N, K, P, D):
    info = plsc.get_sparse_core_info()
    NC, NS = info.num_cores, info.num_subcores
    NW = NC * NS
    NK = N * K
    per_w = NK // NW
    tok = (jnp.arange(NK, dtype=jnp.int32) // K).astype(jnp.int32)
    mesh = plsc.VectorSubcoreMesh(core_axis_name="c", subcore_axis_name="s")

    @functools.partial(
        pl.kernel, mesh=mesh,
        out_type=jax.ShapeDtypeStruct((P, D), jnp.float32),
        scratch_types=[pltpu.VMEM((GW,), jnp.int32),
                       pltpu.VMEM((GW,), jnp.int32),
                       pltpu.VMEM((GW, D), jnp.float32),
                       pltpu.SemaphoreType.DMA])
    def dispatch_kernel(hs_hbm, tok_hbm, pp_hbm, gx_hbm,
                        tok_v, pp_v, rows_v, sem):
        wid = lax.axis_index("s") * NC + lax.axis_index("c")
        base = wid * per_w

        @pl.loop(0, per_w, step=GW)
        def _(c):
            off = base + c
            pltpu.sync_copy(tok_hbm.at[pl.ds(off, GW)], tok_v)
            pltpu.sync_copy(pp_hbm.at[pl.ds(off, GW)], pp_v)
            pltpu.async_copy(hs_hbm.at[tok_v], rows_v, sem).wait()
            pltpu.sync_copy(rows_v, gx_hbm.at[pp_v])

    return dispatch_kernel(hidden_states, tok, pp)


def _tc_grouped_mlp(block_expert, gx, w_up, w_down, N, D, H, E, NB, P,
                    interpret=False):
    HT = H // NJ

    def body(eids_ref, wg_ref, wl_ref, wd_ref, gx_ref, o_ref,
             wg_s, wl_s, wd_s):
        j = pl.program_id(0)
        b = pl.program_id(1)
        new_tile = jnp.logical_or(
            b == 0, eids_ref[b] != eids_ref[jnp.maximum(b - 1, 0)])

        @pl.when(new_tile)
        def _():
            wg_s[...] = wg_ref[0].astype(jnp.bfloat16)
            wl_s[...] = wl_ref[0].astype(jnp.bfloat16)
            wd_s[...] = wd_ref[0].astype(jnp.bfloat16)

        xb = gx_ref[...].astype(jnp.bfloat16)
        hg = lax.dot_general(xb, wg_s[...], (((1,), (1,)), ((), ())),
                             preferred_element_type=jnp.float32)
        hl = lax.dot_general(xb, wl_s[...], (((1,), (1,)), ((), ())),
                             preferred_element_type=jnp.float32)
        g = (hg * jax.nn.sigmoid(hg)) * hl
        part = lax.dot_general(g.astype(jnp.bfloat16), wd_s[...],
                               (((1,), (1,)), ((), ())),
                               preferred_element_type=jnp.float32)
        rows = pl.ds(b * BM, BM)

        @pl.when(j == 0)
        def _():
            o_ref[rows, :] = part

        @pl.when(j > 0)
        def _():
            o_ref[rows, :] = o_ref[rows, :] + part

    grid_spec = pltpu.PrefetchScalarGridSpec(
        num_scalar_prefetch=1,
        grid=(NJ, NB),
        in_specs=[
            pl.BlockSpec((1, HT, D), lambda j, b, eids: (eids[b], j, 0)),
            pl.BlockSpec((1, HT, D), lambda j, b, eids: (eids[b], NJ + j, 0)),
            pl.BlockSpec((1, D, HT), lambda j, b, eids: (eids[b], 0, j)),
            pl.BlockSpec((BM, D), lambda j, b, eids: (b, 0)),
        ],
        out_specs=pl.BlockSpec((P, D), lambda j, b, eids: (0, 0)),
        scratch_shapes=[pltpu.VMEM((HT, D), jnp.bfloat16),
                        pltpu.VMEM((HT, D), jnp.bfloat16),
                        pltpu.VMEM((D, HT), jnp.bfloat16)],
    )
    return pl.pallas_call(
        body,
        grid_spec=grid_spec,
        out_shape=jax.ShapeDtypeStruct((P, D), jnp.float32),
        compiler_params=pltpu.CompilerParams(
            dimension_semantics=("arbitrary", "arbitrary"),
            vmem_limit_bytes=100 * 1024 * 1024,
        ),
        interpret=interpret,
    )(block_expert, w_up, w_up, w_down, gx)


def _sc_combine(y, topk_weights, ppN, N, K, D):
    info = plsc.get_sparse_core_info()
    NC, NS = info.num_cores, info.num_subcores
    NW = NC * NS
    per_w = N // NW
    mesh = plsc.VectorSubcoreMesh(core_axis_name="c", subcore_axis_name="s")
    idx0 = ppN[:, 0]
    idx1 = ppN[:, 1]
    twf = topk_weights.reshape(N * K).astype(jnp.float32)

    cp = pltpu.CompilerParams()
    if "needs_layout_passes" in pltpu.CompilerParams.__dataclass_fields__:
        cp = dataclasses.replace(cp, needs_layout_passes=False)

    @functools.partial(
        pl.kernel, mesh=mesh,
        out_type=jax.ShapeDtypeStruct((N, D), jnp.float32),
        compiler_params=cp,
        scratch_types=[pltpu.VMEM((CW,), jnp.int32),
                       pltpu.VMEM((CW,), jnp.int32),
                       pltpu.VMEM((CW * 2,), jnp.float32),
                       pltpu.VMEM((CW, D), jnp.float32),
                       pltpu.VMEM((CW, D), jnp.float32),
                       pltpu.SemaphoreType.DMA])
    def combine_kernel(y_hbm, tw_hbm, i0_hbm, i1_hbm, out_hbm,
                       i0_v, i1_v, tw_v, acc_v, rows_v, sem):
        wid = lax.axis_index("s") * NC + lax.axis_index("c")
        base = wid * per_w

        @pl.loop(0, per_w, step=CW)
        def _(c):
            off = base + c
            pltpu.sync_copy(i0_hbm.at[pl.ds(off, CW)], i0_v)
            pltpu.sync_copy(i1_hbm.at[pl.ds(off, CW)], i1_v)
            pltpu.sync_copy(tw_hbm.at[pl.ds(2 * off, 2 * CW)], tw_v)
            cp0 = pltpu.async_copy(y_hbm.at[i0_v], acc_v, sem)
            cp1 = pltpu.async_copy(y_hbm.at[i1_v], rows_v, sem)
            cp0.wait()
            cp1.wait()

            @pl.loop(0, CW)
            def _(t):
                tw0 = plsc.load_gather(
                    tw_v, [jnp.full((16,), 2 * t, jnp.int32)])
                tw1 = plsc.load_gather(
                    tw_v, [jnp.full((16,), 2 * t + 1, jnp.int32)])

                @pl.loop(0, D, step=64)
                def _(d):
                    for u in range(4):
                        sl = pl.ds(d + 16 * u, 16)
                        acc_v[t, sl] = (tw0 * acc_v[t, sl]
                                        + tw1 * rows_v[t, sl])

            pltpu.sync_copy(acc_v, out_hbm.at[pl.ds(off, CW)])

    return combine_kernel(y, twf, idx0, idx1)


def kernel(hidden_states, topk_weights, topk_ids, w_up, w_down):
    N, D = hidden_states.shape
    K = topk_ids.shape[1]
    E = w_up.shape[0]
    H = w_down.shape[2]
    NB = (N * K) // BM + E
    P = NB * BM

    block_expert, pp = _routing(topk_ids, N, K, E, NB)
    gx = _sc_dispatch(hidden_states, pp, N, K, P, D)
    y = _tc_grouped_mlp(block_expert, gx, w_up, w_down, N, D, H, E, NB, P)
    return _sc_combine(y, topk_weights, pp.reshape(N, K), N, K, D)
